# Initial kernel scaffold; baseline (speedup 1.0000x reference)
import functools

import jax
import jax.numpy as jnp
from jax import lax
from jax.experimental import pallas as pl
from jax.experimental.pallas import tpu as pltpu
from jax.experimental.pallas import tpu_sc as plsc

N_IDX = 16384 * 50
D = 64
NC = 2
NS = 16
NW = NC * NS
PER_W = N_IDX // NW
CHUNK = 128
N_CHUNKS = PER_W // CHUNK


def _emb_body(idx_hbm, table_hbm, out_hbm, idx_v, rows_v, gsem):
    wid = lax.axis_index("s") * NC + lax.axis_index("c")
    base = wid * PER_W
    pltpu.sync_copy(idx_hbm.at[wid], idx_v)

    def chunk(j, carry):
        pltpu.async_copy(table_hbm.at[idx_v.at[j]], rows_v, gsem).wait()
        pltpu.sync_copy(rows_v, out_hbm.at[pl.ds(base + j * CHUNK, CHUNK)])
        return carry

    lax.fori_loop(0, N_CHUNKS, chunk, 0)


@jax.jit
def _emb_call(idx32, table):
    mesh = plsc.VectorSubcoreMesh(core_axis_name="c", subcore_axis_name="s")
    f = pl.kernel(
        _emb_body,
        out_type=jax.ShapeDtypeStruct((N_IDX, D), jnp.float32),
        mesh=mesh,
        scratch_types=[
            pltpu.VMEM((N_CHUNKS, CHUNK), jnp.int32),
            pltpu.VMEM((CHUNK, D), jnp.float32),
            pltpu.SemaphoreType.DMA,
        ],
    )
    return f(idx32, table)


def kernel(idx, table):
    idx32 = idx.astype(jnp.int32).reshape(NW, N_CHUNKS, CHUNK)
    out = _emb_call(idx32, table)
    return out.reshape(idx.shape[0], idx.shape[1], D)


# speedup vs baseline: 1.6940x; 1.6940x over previous
import functools

import jax
import jax.numpy as jnp
from jax import lax
from jax.experimental import pallas as pl
from jax.experimental.pallas import tpu as pltpu
from jax.experimental.pallas import tpu_sc as plsc

N_IDX = 16384 * 50
D = 64
NC = 2
NS = 16
NW = NC * NS
PER_W = N_IDX // NW
CHUNK = 128
N_CHUNKS = PER_W // CHUNK


def _emb_body(idx_hbm, table_hbm, out_hbm, idx_v, rows_v, gsem):
    wid = lax.axis_index("s") * NC + lax.axis_index("c")
    base = wid * PER_W
    pltpu.sync_copy(idx_hbm.at[wid], idx_v)

    def chunk(j, carry):
        pltpu.async_copy(table_hbm.at[idx_v.at[j]], rows_v, gsem).wait()
        pltpu.sync_copy(rows_v, out_hbm.at[pl.ds(base + j * CHUNK, CHUNK)])
        return carry

    lax.fori_loop(0, N_CHUNKS, chunk, 0)


@jax.jit
def _emb_call(idx32, table):
    mesh = plsc.VectorSubcoreMesh(core_axis_name="c", subcore_axis_name="s")
    f = pl.kernel(
        _emb_body,
        out_type=jax.ShapeDtypeStruct((N_IDX, D), jnp.float32),
        mesh=mesh,
        scratch_types=[
            pltpu.VMEM((N_CHUNKS, CHUNK), jnp.int32),
            pltpu.VMEM((CHUNK, D), jnp.float32),
            pltpu.SemaphoreType.DMA,
        ],
        compiler_params=pltpu.CompilerParams(use_tc_tiling_on_sc=False),
    )
    return f(idx32, table)


def kernel(idx, table):
    idx32 = idx.astype(jnp.int32).reshape(NW, N_CHUNKS, CHUNK)
    out = _emb_call(idx32, table)
    return out.reshape(idx.shape[0], idx.shape[1], D)


# speedup vs baseline: 1.8736x; 1.1060x over previous
import jax
import jax.numpy as jnp
from jax import lax
from jax.experimental import pallas as pl
from jax.experimental.pallas import tpu as pltpu
from jax.experimental.pallas import tpu_sc as plsc

N_IDX = 16384 * 50
D = 64
NC = 2
NS = 16
NW = NC * NS
PER_W = N_IDX // NW
CHUNK = 128
N_CHUNKS = PER_W // CHUNK
NBUF = 8
LOOK = 4
N_GROUPS = N_CHUNKS // NBUF


def _emb_body(idx_hbm, table_hbm, out_hbm, idx_v, rows_v, *sems):
    gsems = sems[:NBUF]
    wsems = sems[NBUF:]
    wid = lax.axis_index("s") * NC + lax.axis_index("c")
    base = wid * PER_W
    pltpu.sync_copy(idx_hbm.at[wid], idx_v)

    def fire_gather(j, b):
        pltpu.async_copy(table_hbm.at[idx_v.at[j]], rows_v.at[b], gsems[b])

    def wait_gather(j, b):
        pltpu.make_async_copy(
            table_hbm.at[idx_v.at[j]], rows_v.at[b], gsems[b]).wait()

    def fire_write(j, b):
        pltpu.async_copy(
            rows_v.at[b], out_hbm.at[pl.ds(base + j * CHUNK, CHUNK)], wsems[b])

    def wait_write(j, b):
        pltpu.make_async_copy(
            rows_v.at[b], out_hbm.at[pl.ds(base + j * CHUNK, CHUNK)],
            wsems[b]).wait()

    for b in range(LOOK):
        fire_gather(b, b)
    for j in range(NBUF):
        b = j % NBUF
        wait_gather(j, b)
        fire_write(j, b)
        b2 = (b + LOOK) % NBUF
        if j >= LOOK:
            wait_write(j - LOOK, b2)
        fire_gather(j + LOOK, b2)

    def group(g, carry):
        j0 = g * NBUF
        for b in range(NBUF):
            j = j0 + b
            wait_gather(j, b)
            fire_write(j, b)
            b2 = (b + LOOK) % NBUF
            wait_write(j - LOOK, b2)
            fire_gather(j + LOOK, b2)
        return carry

    lax.fori_loop(1, N_GROUPS - 1, group, 0)

    j0 = (N_GROUPS - 1) * NBUF
    for b in range(NBUF):
        j = j0 + b
        wait_gather(j, b)
        fire_write(j, b)
        b2 = (b + LOOK) % NBUF
        wait_write(j - LOOK, b2)
        if j + LOOK < N_CHUNKS:
            fire_gather(j + LOOK, b2)
    for b in range(NBUF - LOOK, NBUF):
        wait_write(j0 + b, b)


@jax.jit
def _emb_call(idx32, table):
    mesh = plsc.VectorSubcoreMesh(core_axis_name="c", subcore_axis_name="s")
    f = pl.kernel(
        _emb_body,
        out_type=jax.ShapeDtypeStruct((N_IDX, D), jnp.float32),
        mesh=mesh,
        scratch_types=(
            [pltpu.VMEM((N_CHUNKS, CHUNK), jnp.int32),
             pltpu.VMEM((NBUF, CHUNK, D), jnp.float32)]
            + [pltpu.SemaphoreType.DMA] * (2 * NBUF)
        ),
        compiler_params=pltpu.CompilerParams(use_tc_tiling_on_sc=False),
    )
    return f(idx32, table)


def kernel(idx, table):
    idx32 = idx.astype(jnp.int32).reshape(NW, N_CHUNKS, CHUNK)
    out = _emb_call(idx32, table)
    return out.reshape(idx.shape[0], idx.shape[1], D)
